# baseline (speedup 1.0000x reference)
import jax
import jax.numpy as jnp
from jax import lax
from jax.experimental import pallas as pl
from jax.experimental.pallas import tpu as pltpu
from jax.experimental.pallas import tpu_sc as plsc

N = 10000
E = 320000
H = 128
M = 64
R = 8
SH = 9
B = 16
EPS = 1e-5

NC = 2
NS = 16
NW = NC * NS
EPT = E // NW
CHUNK = 40
NCHUNK = EPT // CHUNK
NP = 10240
RPT = NP // NS
RCH = 40
NRCH = RPT // RCH

_TC1_BLK = 4000
_TC1_GRID = E // _TC1_BLK


def _tc1_body(rbf_ref, W1_ref, b1_ref, W2_ref, b2_ref,
              V1_ref, c1_ref, V2_ref, c2_ref, w_ref, w2_ref):
    r = rbf_ref[...]
    h1 = jax.nn.silu(r @ W1_ref[...] + b1_ref[...][None, :])
    w_ref[...] = h1 @ W2_ref[...] + b2_ref[...][None, :]
    h2 = jax.nn.silu(r @ V1_ref[...] + c1_ref[...][None, :])
    w2_ref[...] = h2 @ V2_ref[...] + c2_ref[...][None, :]


def _tc1(rbf, W1, b1, W2, b2, V1, c1, V2, c2):
    full = lambda shape: pl.BlockSpec(shape, lambda i: tuple(0 for _ in shape))
    return pl.pallas_call(
        _tc1_body,
        grid=(_TC1_GRID,),
        in_specs=[
            pl.BlockSpec((_TC1_BLK, R), lambda i: (i, 0)),
            full((R, M)), full((M,)), full((M, H)), full((H,)),
            full((R, M)), full((M,)), full((M, H)), full((H,)),
        ],
        out_specs=[
            pl.BlockSpec((_TC1_BLK, H), lambda i: (i, 0)),
            pl.BlockSpec((_TC1_BLK, H), lambda i: (i, 0)),
        ],
        out_shape=[
            jax.ShapeDtypeStruct((E, H), jnp.float32),
            jax.ShapeDtypeStruct((E, H), jnp.float32),
        ],
    )(rbf, W1, b1, W2, b2, V1, c1, V2, c2)


def _tcg_body(rbf_ref, sh_ref, Wg_ref, bg_ref, ep0_ref):
    g = rbf_ref[...] @ Wg_ref[...] + bg_ref[...][None, :]
    sh = sh_ref[...] * g
    ep0_ref[...] = jnp.concatenate(
        [sh, jnp.zeros((sh.shape[0], 16 - SH), jnp.float32)], axis=1)


def _tcg(rbf, edge_sh, Wg, bg):
    full = lambda shape: pl.BlockSpec(shape, lambda i: tuple(0 for _ in shape))
    return pl.pallas_call(
        _tcg_body,
        grid=(_TC1_GRID,),
        in_specs=[
            pl.BlockSpec((_TC1_BLK, R), lambda i: (i, 0)),
            pl.BlockSpec((_TC1_BLK, SH), lambda i: (i, 0)),
            full((R, 1)), full((1,)),
        ],
        out_specs=pl.BlockSpec((_TC1_BLK, 16), lambda i: (i, 0)),
        out_shape=jax.ShapeDtypeStruct((E, 16), jnp.float32),
    )(rbf, edge_sh, Wg, bg)


def _pospad_body(p_ref, o_ref):
    p = p_ref[...]
    o_ref[...] = jnp.concatenate(
        [p, jnp.zeros((p.shape[0], H - SH), jnp.float32)], axis=1)


def _pospad(pos):
    blk = 2000
    return pl.pallas_call(
        _pospad_body,
        grid=(N // blk,),
        in_specs=[pl.BlockSpec((blk, SH), lambda i: (i, 0))],
        out_specs=pl.BlockSpec((blk, H), lambda i: (i, 0)),
        out_shape=jax.ShapeDtypeStruct((N, H), jnp.float32),
    )(pos)


def _sc1x_body(x_hbm, src_hbm, dst_hbm, w_hbm, accx_hbm,
               src_v, dst_v, rows_v, w_v, scat_v, accx_sh,
               semg0, semg1):
    cid = lax.axis_index("c")
    sid = lax.axis_index("s")
    wid = sid * NC + cid
    semg = (semg0, semg1)

    zero16 = jnp.zeros((16,), jnp.float32)

    def _zr(i, _):
        r = i // 8
        k = i - r * 8
        scat_v[0, r, pl.ds(k * 16, 16)] = zero16
        return 0
    lax.fori_loop(0, CHUNK * 8, _zr, 0)
    for j in range(NRCH):
        r0 = sid * RPT + j * RCH
        pltpu.sync_copy(scat_v.at[0], accx_sh.at[pl.ds(r0, RCH)])
    plsc.subcore_barrier()

    def _load(c, b):
        base = wid * EPT + c * CHUNK
        pltpu.sync_copy(src_hbm.at[pl.ds(base, CHUNK)], src_v.at[b])
        pltpu.sync_copy(dst_hbm.at[pl.ds(base, CHUNK)], dst_v.at[b])
        pltpu.async_copy(x_hbm.at[src_v.at[b]], rows_v.at[b], semg[b])
        pltpu.async_copy(w_hbm.at[pl.ds(base, CHUNK)], w_v.at[b], semg[b])

    def _drain_g(b):
        pltpu.make_async_copy(
            x_hbm.at[pl.ds(0, CHUNK)], rows_v.at[b], semg[b]).wait()
        pltpu.make_async_copy(
            w_hbm.at[pl.ds(0, CHUNK)], w_v.at[b], semg[b]).wait()

    _load(0, 0)

    def _outer(io, _):
        c0 = io * 2
        for b in range(2):
            c = c0 + b
            nb = 1 - b
            cn = jnp.where(c + 1 < NCHUNK, c + 1, 0)
            _load(cn, nb)
            _drain_g(b)

            def _edge(e, _):
                for k in range(H // 16):
                    scat_v[b, e, pl.ds(k * 16, 16)] = (
                        rows_v[b, e, pl.ds(k * 16, 16)]
                        * w_v[b, e, pl.ds(k * 16, 16)])
                return 0
            lax.fori_loop(0, CHUNK, _edge, 0)

            pltpu.sync_copy(scat_v.at[b], accx_sh.at[dst_v.at[b]], add=True)
        return 0
    lax.fori_loop(0, NCHUNK // 2, _outer, 0)
    _drain_g(0)

    plsc.subcore_barrier()
    for j in range(NRCH):
        r0 = sid * RPT + j * RCH
        pltpu.sync_copy(accx_sh.at[pl.ds(r0, RCH)], scat_v.at[0])
        pltpu.sync_copy(scat_v.at[0], accx_hbm.at[cid, pl.ds(r0, RCH)])


def _sc1x(x, src, dst, w):
    mesh = plsc.VectorSubcoreMesh(core_axis_name="c", subcore_axis_name="s")
    f = pl.kernel(
        _sc1x_body,
        out_type=jax.ShapeDtypeStruct((NC, NP, H), jnp.float32),
        mesh=mesh,
        scratch_types=[
            pltpu.VMEM((2, CHUNK), jnp.int32),
            pltpu.VMEM((2, CHUNK), jnp.int32),
            pltpu.VMEM((2, CHUNK, H), jnp.float32),
            pltpu.VMEM((2, CHUNK, H), jnp.float32),
            pltpu.VMEM((2, CHUNK, H), jnp.float32),
            pltpu.VMEM_SHARED((NP, H), jnp.float32),
            pltpu.SemaphoreType.DMA,
            pltpu.SemaphoreType.DMA,
        ],
    )
    return f(x, src, dst, w)


def _sc1pe_body(pos_hbm, src_hbm, dst_hbm, ep0_hbm,
                accp_hbm, ep_hbm,
                src_v, dst_v, posr_v, ep0_v, epo_v, ep128_v, accp_sh,
                semg0, semg1):
    cid = lax.axis_index("c")
    sid = lax.axis_index("s")
    wid = sid * NC + cid
    semg = (semg0, semg1)

    zero16 = jnp.zeros((16,), jnp.float32)

    def _ze(i, _):
        q = i // 8
        k = i - q * 8
        b = q // CHUNK
        ep128_v[b, q - b * CHUNK, pl.ds(k * 16, 16)] = zero16
        return 0
    lax.fori_loop(0, 2 * CHUNK * 8, _ze, 0)
    for j in range(NRCH):
        r0 = sid * RPT + j * RCH
        pltpu.sync_copy(ep128_v.at[0], accp_sh.at[pl.ds(r0, RCH)])
    plsc.subcore_barrier()

    def _load(c, b):
        base = wid * EPT + c * CHUNK
        pltpu.sync_copy(src_hbm.at[pl.ds(base, CHUNK)], src_v.at[b])
        pltpu.sync_copy(dst_hbm.at[pl.ds(base, CHUNK)], dst_v.at[b])
        pltpu.async_copy(pos_hbm.at[src_v.at[b]], posr_v.at[b], semg[b])
        pltpu.async_copy(ep0_hbm.at[pl.ds(base, CHUNK)], ep0_v.at[b], semg[b])

    def _drain_g(b):
        pltpu.make_async_copy(
            pos_hbm.at[pl.ds(0, CHUNK)], posr_v.at[b], semg[b]).wait()
        pltpu.make_async_copy(
            ep0_hbm.at[pl.ds(0, CHUNK)], ep0_v.at[b], semg[b]).wait()

    _load(0, 0)

    def _outer(io, _):
        c0 = io * 2
        for b in range(2):
            c = c0 + b
            nb = 1 - b
            cn = jnp.where(c + 1 < NCHUNK, c + 1, 0)
            _load(cn, nb)
            _drain_g(b)

            def _edge(e, _):
                v = ep0_v[b, e, :] + posr_v[b, e, pl.ds(0, 16)]
                epo_v[b, e, :] = v
                ep128_v[b, e, pl.ds(0, 16)] = v
                return 0
            lax.fori_loop(0, CHUNK, _edge, 0)

            base = wid * EPT + c * CHUNK
            pltpu.sync_copy(ep128_v.at[b], accp_sh.at[dst_v.at[b]], add=True)
            pltpu.sync_copy(epo_v.at[b], ep_hbm.at[pl.ds(base, CHUNK)])
        return 0
    lax.fori_loop(0, NCHUNK // 2, _outer, 0)
    _drain_g(0)

    plsc.subcore_barrier()
    for j in range(NRCH):
        r0 = sid * RPT + j * RCH
        pltpu.sync_copy(accp_sh.at[pl.ds(r0, RCH)], ep128_v.at[0])
        pltpu.sync_copy(ep128_v.at[0], accp_hbm.at[cid, pl.ds(r0, RCH)])


def _sc1pe(pos_pad, src, dst, ep0):
    mesh = plsc.VectorSubcoreMesh(core_axis_name="c", subcore_axis_name="s")
    f = pl.kernel(
        _sc1pe_body,
        out_type=[
            jax.ShapeDtypeStruct((NC, NP, H), jnp.float32),
            jax.ShapeDtypeStruct((E, 16), jnp.float32),
        ],
        mesh=mesh,
        scratch_types=[
            pltpu.VMEM((2, CHUNK), jnp.int32),
            pltpu.VMEM((2, CHUNK), jnp.int32),
            pltpu.VMEM((2, CHUNK, H), jnp.float32),
            pltpu.VMEM((2, CHUNK, 16), jnp.float32),
            pltpu.VMEM((2, CHUNK, 16), jnp.float32),
            pltpu.VMEM((2, CHUNK, H), jnp.float32),
            pltpu.VMEM_SHARED((NP, H), jnp.float32),
            pltpu.SemaphoreType.DMA,
            pltpu.SemaphoreType.DMA,
        ],
    )
    return f(pos_pad, src, dst, ep0)


def _tc2_body(x_ref, accx_ref, pos_ref, accp_ref, x1_ref, pos1_ref):
    x1_ref[...] = x_ref[...] + accx_ref[0] + accx_ref[1]
    accp = accp_ref[0, :, :SH] + accp_ref[1, :, :SH]
    p1 = pos_ref[...] + accp * (1.0 / 32.0)
    pos1_ref[...] = jnp.concatenate(
        [p1, jnp.zeros((p1.shape[0], 16 - SH), jnp.float32)], axis=1)


def _tc2(x, accx, pos, accp):
    blk = 1000
    return pl.pallas_call(
        _tc2_body,
        grid=(N // blk,),
        in_specs=[
            pl.BlockSpec((blk, H), lambda i: (i, 0)),
            pl.BlockSpec((NC, blk, H), lambda i: (0, i, 0)),
            pl.BlockSpec((blk, SH), lambda i: (i, 0)),
            pl.BlockSpec((NC, blk, H), lambda i: (0, i, 0)),
        ],
        out_specs=[
            pl.BlockSpec((blk, H), lambda i: (i, 0)),
            pl.BlockSpec((blk, 16), lambda i: (i, 0)),
        ],
        out_shape=[
            jax.ShapeDtypeStruct((N, H), jnp.float32),
            jax.ShapeDtypeStruct((N, 16), jnp.float32),
        ],
    )(x, accx, pos, accp)


def _tc2b_body(ep_ref, s2_ref):
    ep = ep_ref[...]
    ones = jnp.ones((16, 16), jnp.float32)
    inv = (ep * ep) @ ones
    s2_ref[...] = jax.nn.sigmoid(inv)


def _tc2b(ep):
    return pl.pallas_call(
        _tc2b_body,
        grid=(_TC1_GRID,),
        in_specs=[pl.BlockSpec((_TC1_BLK, 16), lambda i: (i, 0))],
        out_specs=pl.BlockSpec((_TC1_BLK, 16), lambda i: (i, 0)),
        out_shape=jax.ShapeDtypeStruct((E, 16), jnp.float32),
    )(ep)


def _sc2_body(x1_hbm, src_hbm, dst_hbm, w2_hbm, s2_hbm,
              accm_hbm,
              src_v, dst_v, rows_v, w_v, s2_v, scat_v, accm_sh,
              semg0, semg1):
    cid = lax.axis_index("c")
    sid = lax.axis_index("s")
    wid = sid * NC + cid
    semg = (semg0, semg1)

    zero16 = jnp.zeros((16,), jnp.float32)

    def _zr(i, _):
        r = i // 8
        k = i - r * 8
        scat_v[0, r, pl.ds(k * 16, 16)] = zero16
        return 0
    lax.fori_loop(0, CHUNK * 8, _zr, 0)
    for j in range(NRCH):
        r0 = sid * RPT + j * RCH
        pltpu.sync_copy(scat_v.at[0], accm_sh.at[pl.ds(r0, RCH)])
    plsc.subcore_barrier()

    def _load(c, b):
        base = wid * EPT + c * CHUNK
        pltpu.sync_copy(src_hbm.at[pl.ds(base, CHUNK)], src_v.at[b])
        pltpu.sync_copy(dst_hbm.at[pl.ds(base, CHUNK)], dst_v.at[b])
        pltpu.async_copy(x1_hbm.at[src_v.at[b]], rows_v.at[b], semg[b])
        pltpu.async_copy(w2_hbm.at[pl.ds(base, CHUNK)], w_v.at[b], semg[b])
        pltpu.async_copy(s2_hbm.at[pl.ds(base, CHUNK)], s2_v.at[b], semg[b])

    def _drain_g(b):
        pltpu.make_async_copy(
            x1_hbm.at[pl.ds(0, CHUNK)], rows_v.at[b], semg[b]).wait()
        pltpu.make_async_copy(
            w2_hbm.at[pl.ds(0, CHUNK)], w_v.at[b], semg[b]).wait()
        pltpu.make_async_copy(
            s2_hbm.at[pl.ds(0, CHUNK)], s2_v.at[b], semg[b]).wait()

    _load(0, 0)

    def _outer(io, _):
        c0 = io * 2
        for b in range(2):
            c = c0 + b
            nb = 1 - b
            cn = jnp.where(c + 1 < NCHUNK, c + 1, 0)
            _load(cn, nb)
            _drain_g(b)

            def _edge(e, _):
                sg = s2_v[b, e, :]
                for k in range(H // 16):
                    scat_v[b, e, pl.ds(k * 16, 16)] = (
                        rows_v[b, e, pl.ds(k * 16, 16)]
                        * w_v[b, e, pl.ds(k * 16, 16)] * sg)
                return 0
            lax.fori_loop(0, CHUNK, _edge, 0)

            pltpu.sync_copy(scat_v.at[b], accm_sh.at[dst_v.at[b]], add=True)
        return 0
    lax.fori_loop(0, NCHUNK // 2, _outer, 0)
    _drain_g(0)

    plsc.subcore_barrier()
    for j in range(NRCH):
        r0 = sid * RPT + j * RCH
        pltpu.sync_copy(accm_sh.at[pl.ds(r0, RCH)], scat_v.at[0])
        pltpu.sync_copy(scat_v.at[0], accm_hbm.at[cid, pl.ds(r0, RCH)])


def _sc2(x1, src, dst, w2, s2):
    mesh = plsc.VectorSubcoreMesh(core_axis_name="c", subcore_axis_name="s")
    f = pl.kernel(
        _sc2_body,
        out_type=jax.ShapeDtypeStruct((NC, NP, H), jnp.float32),
        mesh=mesh,
        scratch_types=[
            pltpu.VMEM((2, CHUNK), jnp.int32),
            pltpu.VMEM((2, CHUNK), jnp.int32),
            pltpu.VMEM((2, CHUNK, H), jnp.float32),
            pltpu.VMEM((2, CHUNK, H), jnp.float32),
            pltpu.VMEM((2, CHUNK, 16), jnp.float32),
            pltpu.VMEM((2, CHUNK, H), jnp.float32),
            pltpu.VMEM_SHARED((NP, H), jnp.float32),
            pltpu.SemaphoreType.DMA,
            pltpu.SemaphoreType.DMA,
        ],
    )
    return f(x1, src, dst, w2, s2)


def _tc3_body(x1_ref, accm_ref, pos1_ref, batch_ref, Wo_ref, bo_ref,
              alpha_ref, gamma_ref, beta_ref, bnw_ref, bnb_ref,
              xout_ref, posout_ref):
    x1 = x1_ref[...]
    acc = accm_ref[0, :N, :] + accm_ref[1, :N, :]
    m = acc @ Wo_ref[...] + bo_ref[...][None, :]
    x2 = x1 + m

    b = batch_ref[...]
    iot = lax.broadcasted_iota(jnp.int32, (1, B), 1)
    oh = (b == iot).astype(jnp.float32)
    ones = jnp.ones((N, 1), jnp.float32)
    cnt = jnp.maximum(
        lax.dot_general(oh, ones, (((0,), (0,)), ((), ()))), 1.0)
    mean = lax.dot_general(oh, x2, (((0,), (0,)), ((), ()))) / cnt
    xc = x2 - alpha_ref[...][None, :] * (oh @ mean)
    var = lax.dot_general(oh, xc * xc, (((0,), (0,)), ((), ()))) / cnt
    xout_ref[...] = (gamma_ref[...][None, :] * xc *
                     lax.rsqrt(oh @ var + EPS) + beta_ref[...][None, :])

    p = pos1_ref[...]
    scol = p[:, 0:1]
    smean = jnp.sum(scol) * (1.0 / N)
    svar = jnp.sum((scol - smean) ** 2) * (1.0 / N)
    v1 = p[:, 1:4]
    n1 = jnp.sum(v1 * v1) * (1.0 / (3 * N))
    v2 = p[:, 4:9]
    n2 = jnp.sum(v2 * v2) * (1.0 / (5 * N))
    sc0 = bnw_ref[0, 0] * lax.rsqrt(svar + EPS)
    off0 = bnb_ref[0, 0] - smean * sc0
    sc1 = bnw_ref[0, 1] * lax.rsqrt(n1 + EPS)
    sc2 = bnw_ref[0, 2] * lax.rsqrt(n2 + EPS)
    lane = lax.broadcasted_iota(jnp.int32, (1, 16), 1)
    scale = jnp.where(lane == 0, sc0,
                      jnp.where(lane < 4, sc1,
                                jnp.where(lane < 9, sc2, 0.0)))
    off = jnp.where(lane == 0, off0, 0.0)
    posout_ref[...] = p * scale + off


def _tc3(x1, accm, pos1, batch2, Wo, bo, alpha, gamma, beta, bnw, bnb):
    full = lambda shape: pl.BlockSpec(shape, lambda: tuple(0 for _ in shape))
    return pl.pallas_call(
        _tc3_body,
        in_specs=[
            full((N, H)), full((NC, NP, H)), full((N, 16)), full((N, 1)),
            full((H, H)), full((H,)), full((H,)), full((H,)), full((H,)),
            full((1, 3)), full((1, 1)),
        ],
        out_specs=[full((N, H)), full((N, 16))],
        out_shape=[
            jax.ShapeDtypeStruct((N, H), jnp.float32),
            jax.ShapeDtypeStruct((N, 16), jnp.float32),
        ],
    )(x1, accm, pos1, batch2, Wo, bo, alpha, gamma, beta, bnw, bnb)


def kernel(x, pos, edge_index, rbf, edge_sh, batch, W1, b1, W2, b2, Wg, bg,
           V1, c1, V2, c2, Wo, bo, alpha, gamma, beta, bn_w, bn_b):
    src = edge_index[0]
    dst = edge_index[1]
    batch2 = batch[:, None]
    pos_pad128 = _pospad(pos)

    ep0 = _tcg(rbf, edge_sh, Wg, bg)
    accp, ep = _sc1pe(pos_pad128, src, dst, ep0)
    w, w2 = _tc1(rbf, W1, b1, W2, b2, V1, c1, V2, c2)
    accx = _sc1x(x, src, dst, w)
    s2 = _tc2b(ep)
    x1, pos1 = _tc2(x, accx, pos, accp)
    accm = _sc2(x1, src, dst, w2, s2)
    x_out, pos_out_pad = _tc3(x1, accm, pos1, batch2, Wo, bo, alpha, gamma,
                              beta, bn_w[None, :], bn_b[None, :])
    return x_out, pos_out_pad[:, :SH]


# speedup vs baseline: 1.0448x; 1.0448x over previous
import functools

import jax
import jax.numpy as jnp
from jax import lax
from jax.experimental import pallas as pl
from jax.experimental.pallas import tpu as pltpu
from jax.experimental.pallas import tpu_sc as plsc

N = 10000
E = 320000
H = 128
M = 64
R = 8
SH = 9
B = 16
EPS = 1e-5

NC = 2
NS = 16
NW = NC * NS
EPT = E // NW
CHUNK = 40
NCHUNK = EPT // CHUNK
PCHUNK = 80
PNCHUNK = EPT // PCHUNK
NP = 10240
RPT = NP // NS
RCH = 40
NRCH = RPT // RCH
PRCH = 80
PNRCH = RPT // PRCH

_TC1_BLK = 4000
_TC1_GRID = E // _TC1_BLK


def _tc1a_body(rbf_ref, sh_ref, W1_ref, b1_ref, W2_ref, b2_ref,
               Wg_ref, bg_ref, w_ref, ep0_ref):
    r = rbf_ref[...]
    h1 = jax.nn.silu(r @ W1_ref[...] + b1_ref[...][None, :])
    w_ref[...] = h1 @ W2_ref[...] + b2_ref[...][None, :]
    g = r @ Wg_ref[...] + bg_ref[...][None, :]
    sh = sh_ref[...] * g
    ep0_ref[...] = jnp.concatenate(
        [sh, jnp.zeros((sh.shape[0], 16 - SH), jnp.float32)], axis=1)


def _tc1a(rbf, sh_pad, W1, b1, W2, b2, Wg, bg):
    full = lambda shape: pl.BlockSpec(shape, lambda i: tuple(0 for _ in shape))
    return pl.pallas_call(
        _tc1a_body,
        grid=(_TC1_GRID,),
        in_specs=[
            pl.BlockSpec((_TC1_BLK, R), lambda i: (i, 0)),
            pl.BlockSpec((_TC1_BLK, SH), lambda i: (i, 0)),
            full((R, M)), full((M,)), full((M, H)), full((H,)),
            full((R, 1)), full((1,)),
        ],
        out_specs=[
            pl.BlockSpec((_TC1_BLK, H), lambda i: (i, 0)),
            pl.BlockSpec((_TC1_BLK, 16), lambda i: (i, 0)),
        ],
        out_shape=[
            jax.ShapeDtypeStruct((E, H), jnp.float32),
            jax.ShapeDtypeStruct((E, 16), jnp.float32),
        ],
    )(rbf, sh_pad, W1, b1, W2, b2, Wg, bg)


def _tc1b_body(rbf_ref, V1_ref, c1_ref, V2_ref, c2_ref, w2_ref):
    r = rbf_ref[...]
    h2 = jax.nn.silu(r @ V1_ref[...] + c1_ref[...][None, :])
    w2_ref[...] = h2 @ V2_ref[...] + c2_ref[...][None, :]


def _tc1b(rbf, V1, c1, V2, c2):
    full = lambda shape: pl.BlockSpec(shape, lambda i: tuple(0 for _ in shape))
    return pl.pallas_call(
        _tc1b_body,
        grid=(_TC1_GRID,),
        in_specs=[
            pl.BlockSpec((_TC1_BLK, R), lambda i: (i, 0)),
            full((R, M)), full((M,)), full((M, H)), full((H,)),
        ],
        out_specs=pl.BlockSpec((_TC1_BLK, H), lambda i: (i, 0)),
        out_shape=jax.ShapeDtypeStruct((E, H), jnp.float32),
    )(rbf, V1, c1, V2, c2)


def _pospad_body(p_ref, o_ref):
    p = p_ref[...]
    o_ref[...] = jnp.concatenate(
        [p, jnp.zeros((p.shape[0], H - SH), jnp.float32)], axis=1)


def _pospad(pos):
    blk = 2000
    return pl.pallas_call(
        _pospad_body,
        grid=(N // blk,),
        in_specs=[pl.BlockSpec((blk, SH), lambda i: (i, 0))],
        out_specs=pl.BlockSpec((blk, H), lambda i: (i, 0)),
        out_shape=jax.ShapeDtypeStruct((N, H), jnp.float32),
    )(pos)


def _sc1_body(x_hbm, pos_hbm, src_hbm, dst_hbm, w_hbm, ep0_hbm,
              accx_hbm, ep_hbm,
              src_v, dst_v, rows_v, posr_v, w_v, ep_v,
              accx_sh, sem0, sem1):
    cid = lax.axis_index("c")
    sid = lax.axis_index("s")
    wid = sid * NC + cid
    sems = (sem0, sem1)

    zero16 = jnp.zeros((16,), jnp.float32)

    def _zr(i, _):
        r = i // 8
        k = i - r * 8
        rows_v[0, r, pl.ds(k * 16, 16)] = zero16
        return 0
    lax.fori_loop(0, CHUNK * 8, _zr, 0)

    for j in range(NRCH):
        r0 = sid * RPT + j * RCH
        pltpu.sync_copy(rows_v.at[0], accx_sh.at[pl.ds(r0, RCH)])
    plsc.subcore_barrier()

    def _load(c, b):
        base = wid * EPT + c * CHUNK
        pltpu.sync_copy(src_hbm.at[pl.ds(base, CHUNK)], src_v.at[b])
        pltpu.sync_copy(dst_hbm.at[pl.ds(base, CHUNK)], dst_v.at[b])
        pltpu.async_copy(x_hbm.at[src_v.at[b]], rows_v.at[b], sems[b])
        pltpu.async_copy(pos_hbm.at[src_v.at[b]], posr_v.at[b], sems[b])
        pltpu.async_copy(w_hbm.at[pl.ds(base, CHUNK)], w_v.at[b], sems[b])
        pltpu.async_copy(ep0_hbm.at[pl.ds(base, CHUNK)], ep_v.at[b], sems[b])

    def _drain(b):
        pltpu.make_async_copy(
            x_hbm.at[pl.ds(0, CHUNK)], rows_v.at[b], sems[b]).wait()
        pltpu.make_async_copy(
            pos_hbm.at[pl.ds(0, CHUNK)], posr_v.at[b], sems[b]).wait()
        pltpu.make_async_copy(
            w_hbm.at[pl.ds(0, CHUNK)], w_v.at[b], sems[b]).wait()
        pltpu.make_async_copy(
            ep0_hbm.at[pl.ds(0, CHUNK)], ep_v.at[b], sems[b]).wait()

    _load(0, 0)

    def _outer(io, _):
        c0 = io * 2
        for b in range(2):
            c = c0 + b
            nb = 1 - b
            cn = jnp.where(c + 1 < NCHUNK, c + 1, 0)
            _load(cn, nb)
            _drain(b)

            def _edge(e, _):
                ep_v[b, e, :] = ep_v[b, e, :] + posr_v[b, e, pl.ds(0, 16)]
                for k in range(H // 16):
                    rows_v[b, e, pl.ds(k * 16, 16)] = (
                        rows_v[b, e, pl.ds(k * 16, 16)]
                        * w_v[b, e, pl.ds(k * 16, 16)])
                return 0
            lax.fori_loop(0, CHUNK, _edge, 0)

            base = wid * EPT + c * CHUNK
            pltpu.sync_copy(rows_v.at[b], accx_sh.at[dst_v.at[b]], add=True)
            pltpu.sync_copy(ep_v.at[b], ep_hbm.at[pl.ds(base, CHUNK)])
        return 0
    lax.fori_loop(0, NCHUNK // 2, _outer, 0)
    _drain(0)

    plsc.subcore_barrier()
    for j in range(NRCH):
        r0 = sid * RPT + j * RCH
        pltpu.sync_copy(accx_sh.at[pl.ds(r0, RCH)], rows_v.at[0])
        pltpu.sync_copy(rows_v.at[0], accx_hbm.at[cid, pl.ds(r0, RCH)])


def _sc1(x, pos_pad, src, dst, w, ep0):
    mesh = plsc.VectorSubcoreMesh(core_axis_name="c", subcore_axis_name="s")
    f = pl.kernel(
        _sc1_body,
        out_type=[
            jax.ShapeDtypeStruct((NC, NP, H), jnp.float32),
            jax.ShapeDtypeStruct((E, 16), jnp.float32),
        ],
        mesh=mesh,
        scratch_types=[
            pltpu.VMEM((2, CHUNK), jnp.int32),
            pltpu.VMEM((2, CHUNK), jnp.int32),
            pltpu.VMEM((2, CHUNK, H), jnp.float32),
            pltpu.VMEM((2, CHUNK, H), jnp.float32),
            pltpu.VMEM((2, CHUNK, H), jnp.float32),
            pltpu.VMEM((2, CHUNK, 16), jnp.float32),
            pltpu.VMEM_SHARED((NP, H), jnp.float32),
            pltpu.SemaphoreType.DMA,
            pltpu.SemaphoreType.DMA,
        ],
    )
    return f(x, pos_pad, src, dst, w, ep0)


def _sc1p_body(dst_hbm, ep_hbm, accp_hbm,
               dst_v, ep16_v, ep128_v, accp_sh, sem0, sem1):
    cid = lax.axis_index("c")
    sid = lax.axis_index("s")
    wid = sid * NC + cid
    sems = (sem0, sem1)

    zero16 = jnp.zeros((16,), jnp.float32)

    def _ze(i, _):
        r = i // 8
        k = i - r * 8
        b = r // PCHUNK
        ep128_v[b, r - b * PCHUNK, pl.ds(k * 16, 16)] = zero16
        return 0
    lax.fori_loop(0, 2 * PCHUNK * 8, _ze, 0)

    for j in range(PNRCH):
        r0 = sid * RPT + j * PRCH
        pltpu.sync_copy(ep128_v.at[0], accp_sh.at[pl.ds(r0, PRCH)])
    plsc.subcore_barrier()

    def _load(c, b):
        base = wid * EPT + c * PCHUNK
        pltpu.async_copy(dst_hbm.at[pl.ds(base, PCHUNK)], dst_v.at[b], sems[b])
        pltpu.async_copy(ep_hbm.at[pl.ds(base, PCHUNK)], ep16_v.at[b], sems[b])

    def _drain(b):
        pltpu.make_async_copy(
            dst_hbm.at[pl.ds(0, PCHUNK)], dst_v.at[b], sems[b]).wait()
        pltpu.make_async_copy(
            ep_hbm.at[pl.ds(0, PCHUNK)], ep16_v.at[b], sems[b]).wait()

    _load(0, 0)

    def _outer(io, _):
        c0 = io * 2
        for b in range(2):
            c = c0 + b
            nb = 1 - b
            cn = jnp.where(c + 1 < PNCHUNK, c + 1, 0)
            _load(cn, nb)
            _drain(b)

            def _edge(e, _):
                ep128_v[b, e, pl.ds(0, 16)] = ep16_v[b, e, :]
                return 0
            lax.fori_loop(0, PCHUNK, _edge, 0)

            pltpu.sync_copy(ep128_v.at[b], accp_sh.at[dst_v.at[b]], add=True)
        return 0
    lax.fori_loop(0, PNCHUNK // 2, _outer, 0)
    _drain(0)

    def _edge_last(e, _):
        ep128_v[0, e, pl.ds(0, 16)] = ep16_v[0, e, :]
        return 0
    lax.fori_loop(0, PCHUNK, _edge_last, 0)
    pltpu.sync_copy(ep128_v.at[0],
                    accp_sh.at[dst_v.at[0]], add=True)

    plsc.subcore_barrier()
    for j in range(PNRCH):
        r0 = sid * RPT + j * PRCH
        pltpu.sync_copy(accp_sh.at[pl.ds(r0, PRCH)], ep128_v.at[0])
        pltpu.sync_copy(ep128_v.at[0], accp_hbm.at[cid, pl.ds(r0, PRCH)])


def _sc1p(dst, ep):
    mesh = plsc.VectorSubcoreMesh(core_axis_name="c", subcore_axis_name="s")
    f = pl.kernel(
        _sc1p_body,
        out_type=jax.ShapeDtypeStruct((NC, NP, H), jnp.float32),
        mesh=mesh,
        scratch_types=[
            pltpu.VMEM((2, PCHUNK), jnp.int32),
            pltpu.VMEM((2, PCHUNK, 16), jnp.float32),
            pltpu.VMEM((2, PCHUNK, H), jnp.float32),
            pltpu.VMEM_SHARED((NP, H), jnp.float32),
            pltpu.SemaphoreType.DMA,
            pltpu.SemaphoreType.DMA,
        ],
    )
    return f(dst, ep)


def _tc2_body(x_ref, accx_ref, pos_ref, accp_ref, x1_ref, pos1_ref):
    x1_ref[...] = x_ref[...] + accx_ref[0] + accx_ref[1]
    accp = accp_ref[0, :, :SH] + accp_ref[1, :, :SH]
    p1 = pos_ref[...] + accp * (1.0 / 32.0)
    pos1_ref[...] = jnp.concatenate(
        [p1, jnp.zeros((p1.shape[0], 16 - SH), jnp.float32)], axis=1)


def _tc2(x, accx, pos_pad, accp):
    blk = 1000
    return pl.pallas_call(
        _tc2_body,
        grid=(N // blk,),
        in_specs=[
            pl.BlockSpec((blk, H), lambda i: (i, 0)),
            pl.BlockSpec((NC, blk, H), lambda i: (0, i, 0)),
            pl.BlockSpec((blk, SH), lambda i: (i, 0)),
            pl.BlockSpec((NC, blk, H), lambda i: (0, i, 0)),
        ],
        out_specs=[
            pl.BlockSpec((blk, H), lambda i: (i, 0)),
            pl.BlockSpec((blk, 16), lambda i: (i, 0)),
        ],
        out_shape=[
            jax.ShapeDtypeStruct((N, H), jnp.float32),
            jax.ShapeDtypeStruct((N, 16), jnp.float32),
        ],
    )(x, accx, pos_pad, accp)


def _tc2b_body(ep_ref, s2_ref):
    ep = ep_ref[...]
    ones = jnp.ones((16, 16), jnp.float32)
    inv = (ep * ep) @ ones
    s2_ref[...] = jax.nn.sigmoid(inv)


def _tc2b(ep):
    return pl.pallas_call(
        _tc2b_body,
        grid=(_TC1_GRID,),
        in_specs=[pl.BlockSpec((_TC1_BLK, 16), lambda i: (i, 0))],
        out_specs=pl.BlockSpec((_TC1_BLK, 16), lambda i: (i, 0)),
        out_shape=jax.ShapeDtypeStruct((E, 16), jnp.float32),
    )(ep)


def _sc2_body(x1_hbm, src_hbm, dst_hbm, w2_hbm, s2_hbm,
              accm_hbm,
              src_v, dst_v, rows_v, w_v, s2_v, accm_sh, sem0, sem1):
    cid = lax.axis_index("c")
    sid = lax.axis_index("s")
    wid = sid * NC + cid
    sems = (sem0, sem1)

    zero16 = jnp.zeros((16,), jnp.float32)

    def _zr(i, _):
        r = i // 8
        k = i - r * 8
        rows_v[0, r, pl.ds(k * 16, 16)] = zero16
        return 0
    lax.fori_loop(0, CHUNK * 8, _zr, 0)

    for j in range(NRCH):
        r0 = sid * RPT + j * RCH
        pltpu.sync_copy(rows_v.at[0], accm_sh.at[pl.ds(r0, RCH)])
    plsc.subcore_barrier()

    def _load(c, b):
        base = wid * EPT + c * CHUNK
        pltpu.sync_copy(src_hbm.at[pl.ds(base, CHUNK)], src_v.at[b])
        pltpu.sync_copy(dst_hbm.at[pl.ds(base, CHUNK)], dst_v.at[b])
        pltpu.async_copy(x1_hbm.at[src_v.at[b]], rows_v.at[b], sems[b])
        pltpu.async_copy(w2_hbm.at[pl.ds(base, CHUNK)], w_v.at[b], sems[b])
        pltpu.async_copy(s2_hbm.at[pl.ds(base, CHUNK)], s2_v.at[b], sems[b])

    def _drain(b):
        pltpu.make_async_copy(
            x1_hbm.at[pl.ds(0, CHUNK)], rows_v.at[b], sems[b]).wait()
        pltpu.make_async_copy(
            w2_hbm.at[pl.ds(0, CHUNK)], w_v.at[b], sems[b]).wait()
        pltpu.make_async_copy(
            s2_hbm.at[pl.ds(0, CHUNK)], s2_v.at[b], sems[b]).wait()

    _load(0, 0)

    def _outer(io, _):
        c0 = io * 2
        for b in range(2):
            c = c0 + b
            nb = 1 - b
            cn = jnp.where(c + 1 < NCHUNK, c + 1, 0)
            _load(cn, nb)
            _drain(b)

            def _edge(e, _):
                sg = s2_v[b, e, :]
                for k in range(H // 16):
                    rows_v[b, e, pl.ds(k * 16, 16)] = (
                        rows_v[b, e, pl.ds(k * 16, 16)]
                        * w_v[b, e, pl.ds(k * 16, 16)] * sg)
                return 0
            lax.fori_loop(0, CHUNK, _edge, 0)

            pltpu.sync_copy(rows_v.at[b], accm_sh.at[dst_v.at[b]], add=True)
        return 0
    lax.fori_loop(0, NCHUNK // 2, _outer, 0)
    _drain(0)

    plsc.subcore_barrier()
    for j in range(NRCH):
        r0 = sid * RPT + j * RCH
        pltpu.sync_copy(accm_sh.at[pl.ds(r0, RCH)], rows_v.at[0])
        pltpu.sync_copy(rows_v.at[0], accm_hbm.at[cid, pl.ds(r0, RCH)])


def _sc2(x1, src, dst, w2, s2):
    mesh = plsc.VectorSubcoreMesh(core_axis_name="c", subcore_axis_name="s")
    f = pl.kernel(
        _sc2_body,
        out_type=jax.ShapeDtypeStruct((NC, NP, H), jnp.float32),
        mesh=mesh,
        scratch_types=[
            pltpu.VMEM((2, CHUNK), jnp.int32),
            pltpu.VMEM((2, CHUNK), jnp.int32),
            pltpu.VMEM((2, CHUNK, H), jnp.float32),
            pltpu.VMEM((2, CHUNK, H), jnp.float32),
            pltpu.VMEM((2, CHUNK, 16), jnp.float32),
            pltpu.VMEM_SHARED((NP, H), jnp.float32),
            pltpu.SemaphoreType.DMA,
            pltpu.SemaphoreType.DMA,
        ],
    )
    return f(x1, src, dst, w2, s2)


def _tc3_body(x1_ref, accm_ref, pos1_ref, batch_ref, Wo_ref, bo_ref,
              alpha_ref, gamma_ref, beta_ref, bnw_ref, bnb_ref,
              xout_ref, posout_ref):
    x1 = x1_ref[...]
    acc = accm_ref[0, :N, :] + accm_ref[1, :N, :]
    m = acc @ Wo_ref[...] + bo_ref[...][None, :]
    x2 = x1 + m

    b = batch_ref[...]
    iot = lax.broadcasted_iota(jnp.int32, (1, B), 1)
    oh = (b == iot).astype(jnp.float32)
    ones = jnp.ones((N, 1), jnp.float32)
    cnt = jnp.maximum(
        lax.dot_general(oh, ones, (((0,), (0,)), ((), ()))), 1.0)
    mean = lax.dot_general(oh, x2, (((0,), (0,)), ((), ()))) / cnt
    xc = x2 - alpha_ref[...][None, :] * (oh @ mean)
    var = lax.dot_general(oh, xc * xc, (((0,), (0,)), ((), ()))) / cnt
    xout_ref[...] = (gamma_ref[...][None, :] * xc *
                     lax.rsqrt(oh @ var + EPS) + beta_ref[...][None, :])

    p = pos1_ref[...]
    scol = p[:, 0:1]
    smean = jnp.sum(scol) * (1.0 / N)
    svar = jnp.sum((scol - smean) ** 2) * (1.0 / N)
    v1 = p[:, 1:4]
    n1 = jnp.sum(v1 * v1) * (1.0 / (3 * N))
    v2 = p[:, 4:9]
    n2 = jnp.sum(v2 * v2) * (1.0 / (5 * N))
    sc0 = bnw_ref[0, 0] * lax.rsqrt(svar + EPS)
    off0 = bnb_ref[0, 0] - smean * sc0
    sc1 = bnw_ref[0, 1] * lax.rsqrt(n1 + EPS)
    sc2 = bnw_ref[0, 2] * lax.rsqrt(n2 + EPS)
    lane = lax.broadcasted_iota(jnp.int32, (1, 16), 1)
    scale = jnp.where(lane == 0, sc0,
                      jnp.where(lane < 4, sc1,
                                jnp.where(lane < 9, sc2, 0.0)))
    off = jnp.where(lane == 0, off0, 0.0)
    posout_ref[...] = p * scale + off


def _tc3(x1, accm, pos1, batch2, Wo, bo, alpha, gamma, beta, bnw, bnb):
    full = lambda shape: pl.BlockSpec(shape, lambda: tuple(0 for _ in shape))
    return pl.pallas_call(
        _tc3_body,
        in_specs=[
            full((N, H)), full((NC, NP, H)), full((N, 16)), full((N, 1)),
            full((H, H)), full((H,)), full((H,)), full((H,)), full((H,)),
            full((1, 3)), full((1, 1)),
        ],
        out_specs=[full((N, H)), full((N, 16))],
        out_shape=[
            jax.ShapeDtypeStruct((N, H), jnp.float32),
            jax.ShapeDtypeStruct((N, 16), jnp.float32),
        ],
    )(x1, accm, pos1, batch2, Wo, bo, alpha, gamma, beta, bnw, bnb)


def kernel(x, pos, edge_index, rbf, edge_sh, batch, W1, b1, W2, b2, Wg, bg,
           V1, c1, V2, c2, Wo, bo, alpha, gamma, beta, bn_w, bn_b):
    src = edge_index[0]
    dst = edge_index[1]
    batch2 = batch[:, None]
    pos_pad128 = _pospad(pos)

    w, ep0 = _tc1a(rbf, edge_sh, W1, b1, W2, b2, Wg, bg)
    w2 = _tc1b(rbf, V1, c1, V2, c2)
    accx, ep = _sc1(x, pos_pad128, src, dst, w, ep0)
    accp = _sc1p(dst, ep)
    s2 = _tc2b(ep)
    x1, pos1 = _tc2(x, accx, pos, accp)
    accm = _sc2(x1, src, dst, w2, s2)
    x_out, pos_out_pad = _tc3(x1, accm, pos1, batch2, Wo, bo, alpha, gamma,
                              beta, bn_w[None, :], bn_b[None, :])
    return x_out, pos_out_pad[:, :SH]


# speedup vs baseline: 1.0633x; 1.0177x over previous
import functools

import jax
import jax.numpy as jnp
from jax import lax
from jax.experimental import pallas as pl
from jax.experimental.pallas import tpu as pltpu
from jax.experimental.pallas import tpu_sc as plsc

N = 10000
E = 320000
H = 128
M = 64
R = 8
SH = 9
B = 16
EPS = 1e-5

NC = 2
NS = 16
NW = NC * NS
EPT = E // NW
CHUNK = 40
NCHUNK = EPT // CHUNK
PCHUNK = 80
PNCHUNK = EPT // PCHUNK
NP = 10240
RPT = NP // NS
RCH = 40
NRCH = RPT // RCH
PRCH = 80
PNRCH = RPT // PRCH

_TC1_BLK = 8000
_TC1_GRID = E // _TC1_BLK


def _tc1a_body(rbf_ref, sh_ref, W1_ref, b1_ref, W2_ref, b2_ref,
               Wg_ref, bg_ref, w_ref, ep0_ref):
    r = rbf_ref[...]
    h1 = jax.nn.silu(r @ W1_ref[...] + b1_ref[...][None, :])
    w_ref[...] = h1 @ W2_ref[...] + b2_ref[...][None, :]
    g = r @ Wg_ref[...] + bg_ref[...][None, :]
    sh = sh_ref[...] * g
    ep0_ref[...] = jnp.concatenate(
        [sh, jnp.zeros((sh.shape[0], 16 - SH), jnp.float32)], axis=1)


def _tc1a(rbf, sh_pad, W1, b1, W2, b2, Wg, bg):
    full = lambda shape: pl.BlockSpec(shape, lambda i: tuple(0 for _ in shape))
    return pl.pallas_call(
        _tc1a_body,
        grid=(_TC1_GRID,),
        in_specs=[
            pl.BlockSpec((_TC1_BLK, R), lambda i: (i, 0)),
            pl.BlockSpec((_TC1_BLK, SH), lambda i: (i, 0)),
            full((R, M)), full((M,)), full((M, H)), full((H,)),
            full((R, 1)), full((1,)),
        ],
        out_specs=[
            pl.BlockSpec((_TC1_BLK, H), lambda i: (i, 0)),
            pl.BlockSpec((_TC1_BLK, 16), lambda i: (i, 0)),
        ],
        out_shape=[
            jax.ShapeDtypeStruct((E, H), jnp.float32),
            jax.ShapeDtypeStruct((E, 16), jnp.float32),
        ],
    )(rbf, sh_pad, W1, b1, W2, b2, Wg, bg)


def _tc1b_body(rbf_ref, V1_ref, c1_ref, V2_ref, c2_ref, w2_ref):
    r = rbf_ref[...]
    h2 = jax.nn.silu(r @ V1_ref[...] + c1_ref[...][None, :])
    w2_ref[...] = h2 @ V2_ref[...] + c2_ref[...][None, :]


def _tc1b(rbf, V1, c1, V2, c2):
    full = lambda shape: pl.BlockSpec(shape, lambda i: tuple(0 for _ in shape))
    return pl.pallas_call(
        _tc1b_body,
        grid=(_TC1_GRID,),
        in_specs=[
            pl.BlockSpec((_TC1_BLK, R), lambda i: (i, 0)),
            full((R, M)), full((M,)), full((M, H)), full((H,)),
        ],
        out_specs=pl.BlockSpec((_TC1_BLK, H), lambda i: (i, 0)),
        out_shape=jax.ShapeDtypeStruct((E, H), jnp.float32),
    )(rbf, V1, c1, V2, c2)


def _pospad_body(p_ref, o_ref):
    p = p_ref[...]
    o_ref[...] = jnp.concatenate(
        [p, jnp.zeros((p.shape[0], H - SH), jnp.float32)], axis=1)


def _pospad(pos):
    blk = 2000
    return pl.pallas_call(
        _pospad_body,
        grid=(N // blk,),
        in_specs=[pl.BlockSpec((blk, SH), lambda i: (i, 0))],
        out_specs=pl.BlockSpec((blk, H), lambda i: (i, 0)),
        out_shape=jax.ShapeDtypeStruct((N, H), jnp.float32),
    )(pos)


def _sc1_body(x_hbm, pos_hbm, src_hbm, dst_hbm, w_hbm, ep0_hbm,
              accx_hbm, ep_hbm,
              src_v, dst_v, rows_v, posr_v, w_v, ep_v,
              accx_sh, sem0, sem1):
    cid = lax.axis_index("c")
    sid = lax.axis_index("s")
    wid = sid * NC + cid
    sems = (sem0, sem1)

    zero16 = jnp.zeros((16,), jnp.float32)

    def _zr(i, _):
        r = i // 8
        k = i - r * 8
        rows_v[0, r, pl.ds(k * 16, 16)] = zero16
        return 0
    lax.fori_loop(0, CHUNK * 8, _zr, 0)

    for j in range(NRCH):
        r0 = sid * RPT + j * RCH
        pltpu.sync_copy(rows_v.at[0], accx_sh.at[pl.ds(r0, RCH)])
    plsc.subcore_barrier()

    def _load(c, b):
        base = wid * EPT + c * CHUNK
        pltpu.sync_copy(src_hbm.at[pl.ds(base, CHUNK)], src_v.at[b])
        pltpu.sync_copy(dst_hbm.at[pl.ds(base, CHUNK)], dst_v.at[b])
        pltpu.async_copy(x_hbm.at[src_v.at[b]], rows_v.at[b], sems[b])
        pltpu.async_copy(pos_hbm.at[src_v.at[b]], posr_v.at[b], sems[b])
        pltpu.async_copy(w_hbm.at[pl.ds(base, CHUNK)], w_v.at[b], sems[b])
        pltpu.async_copy(ep0_hbm.at[pl.ds(base, CHUNK)], ep_v.at[b], sems[b])

    def _drain(b):
        pltpu.make_async_copy(
            x_hbm.at[pl.ds(0, CHUNK)], rows_v.at[b], sems[b]).wait()
        pltpu.make_async_copy(
            pos_hbm.at[pl.ds(0, CHUNK)], posr_v.at[b], sems[b]).wait()
        pltpu.make_async_copy(
            w_hbm.at[pl.ds(0, CHUNK)], w_v.at[b], sems[b]).wait()
        pltpu.make_async_copy(
            ep0_hbm.at[pl.ds(0, CHUNK)], ep_v.at[b], sems[b]).wait()

    _load(0, 0)

    def _outer(io, _):
        c0 = io * 2
        for b in range(2):
            c = c0 + b
            nb = 1 - b
            cn = jnp.where(c + 1 < NCHUNK, c + 1, 0)
            _load(cn, nb)
            _drain(b)

            def _edge(e, _):
                ep_v[b, e, :] = ep_v[b, e, :] + posr_v[b, e, pl.ds(0, 16)]
                for k in range(H // 16):
                    rows_v[b, e, pl.ds(k * 16, 16)] = (
                        rows_v[b, e, pl.ds(k * 16, 16)]
                        * w_v[b, e, pl.ds(k * 16, 16)])
                return 0
            lax.fori_loop(0, CHUNK, _edge, 0)

            base = wid * EPT + c * CHUNK
            pltpu.sync_copy(rows_v.at[b], accx_sh.at[dst_v.at[b]], add=True)
            pltpu.sync_copy(ep_v.at[b], ep_hbm.at[pl.ds(base, CHUNK)])
        return 0
    lax.fori_loop(0, NCHUNK // 2, _outer, 0)
    _drain(0)

    plsc.subcore_barrier()
    for j in range(NRCH):
        r0 = sid * RPT + j * RCH
        pltpu.sync_copy(accx_sh.at[pl.ds(r0, RCH)], rows_v.at[0])
        pltpu.sync_copy(rows_v.at[0], accx_hbm.at[cid, pl.ds(r0, RCH)])


def _sc1(x, pos_pad, src, dst, w, ep0):
    mesh = plsc.VectorSubcoreMesh(core_axis_name="c", subcore_axis_name="s")
    f = pl.kernel(
        _sc1_body,
        out_type=[
            jax.ShapeDtypeStruct((NC, NP, H), jnp.float32),
            jax.ShapeDtypeStruct((E, 16), jnp.float32),
        ],
        mesh=mesh,
        scratch_types=[
            pltpu.VMEM((2, CHUNK), jnp.int32),
            pltpu.VMEM((2, CHUNK), jnp.int32),
            pltpu.VMEM((2, CHUNK, H), jnp.float32),
            pltpu.VMEM((2, CHUNK, H), jnp.float32),
            pltpu.VMEM((2, CHUNK, H), jnp.float32),
            pltpu.VMEM((2, CHUNK, 16), jnp.float32),
            pltpu.VMEM_SHARED((NP, H), jnp.float32),
            pltpu.SemaphoreType.DMA,
            pltpu.SemaphoreType.DMA,
        ],
    )
    return f(x, pos_pad, src, dst, w, ep0)


def _sc1p_body(dst_hbm, ep_hbm, accp_hbm,
               dst_v, ep16_v, ep128_v, accp_sh, sem0, sem1):
    cid = lax.axis_index("c")
    sid = lax.axis_index("s")
    wid = sid * NC + cid
    sems = (sem0, sem1)

    zero16 = jnp.zeros((16,), jnp.float32)

    def _ze(i, _):
        r = i // 8
        k = i - r * 8
        b = r // PCHUNK
        ep128_v[b, r - b * PCHUNK, pl.ds(k * 16, 16)] = zero16
        return 0
    lax.fori_loop(0, 2 * PCHUNK * 8, _ze, 0)

    for j in range(PNRCH):
        r0 = sid * RPT + j * PRCH
        pltpu.sync_copy(ep128_v.at[0], accp_sh.at[pl.ds(r0, PRCH)])
    plsc.subcore_barrier()

    def _load(c, b):
        base = wid * EPT + c * PCHUNK
        pltpu.async_copy(dst_hbm.at[pl.ds(base, PCHUNK)], dst_v.at[b], sems[b])
        pltpu.async_copy(ep_hbm.at[pl.ds(base, PCHUNK)], ep16_v.at[b], sems[b])

    def _drain(b):
        pltpu.make_async_copy(
            dst_hbm.at[pl.ds(0, PCHUNK)], dst_v.at[b], sems[b]).wait()
        pltpu.make_async_copy(
            ep_hbm.at[pl.ds(0, PCHUNK)], ep16_v.at[b], sems[b]).wait()

    _load(0, 0)

    def _outer(io, _):
        c0 = io * 2
        for b in range(2):
            c = c0 + b
            nb = 1 - b
            cn = jnp.where(c + 1 < PNCHUNK, c + 1, 0)
            _load(cn, nb)
            _drain(b)

            def _edge(e, _):
                ep128_v[b, e, pl.ds(0, 16)] = ep16_v[b, e, :]
                return 0
            lax.fori_loop(0, PCHUNK, _edge, 0)

            pltpu.sync_copy(ep128_v.at[b], accp_sh.at[dst_v.at[b]], add=True)
        return 0
    lax.fori_loop(0, PNCHUNK // 2, _outer, 0)
    _drain(0)

    def _edge_last(e, _):
        ep128_v[0, e, pl.ds(0, 16)] = ep16_v[0, e, :]
        return 0
    lax.fori_loop(0, PCHUNK, _edge_last, 0)
    pltpu.sync_copy(ep128_v.at[0],
                    accp_sh.at[dst_v.at[0]], add=True)

    plsc.subcore_barrier()
    for j in range(PNRCH):
        r0 = sid * RPT + j * PRCH
        pltpu.sync_copy(accp_sh.at[pl.ds(r0, PRCH)], ep128_v.at[0])
        pltpu.sync_copy(ep128_v.at[0], accp_hbm.at[cid, pl.ds(r0, PRCH)])


def _sc1p(dst, ep):
    mesh = plsc.VectorSubcoreMesh(core_axis_name="c", subcore_axis_name="s")
    f = pl.kernel(
        _sc1p_body,
        out_type=jax.ShapeDtypeStruct((NC, NP, H), jnp.float32),
        mesh=mesh,
        scratch_types=[
            pltpu.VMEM((2, PCHUNK), jnp.int32),
            pltpu.VMEM((2, PCHUNK, 16), jnp.float32),
            pltpu.VMEM((2, PCHUNK, H), jnp.float32),
            pltpu.VMEM_SHARED((NP, H), jnp.float32),
            pltpu.SemaphoreType.DMA,
            pltpu.SemaphoreType.DMA,
        ],
    )
    return f(dst, ep)


def _tc2_body(x_ref, accx_ref, pos_ref, accp_ref, x1_ref, pos1_ref):
    x1_ref[...] = x_ref[...] + accx_ref[0] + accx_ref[1]
    accp = accp_ref[0, :, :SH] + accp_ref[1, :, :SH]
    p1 = pos_ref[...] + accp * (1.0 / 32.0)
    pos1_ref[...] = jnp.concatenate(
        [p1, jnp.zeros((p1.shape[0], 16 - SH), jnp.float32)], axis=1)


def _tc2(x, accx, pos_pad, accp):
    blk = 1000
    return pl.pallas_call(
        _tc2_body,
        grid=(N // blk,),
        in_specs=[
            pl.BlockSpec((blk, H), lambda i: (i, 0)),
            pl.BlockSpec((NC, blk, H), lambda i: (0, i, 0)),
            pl.BlockSpec((blk, SH), lambda i: (i, 0)),
            pl.BlockSpec((NC, blk, H), lambda i: (0, i, 0)),
        ],
        out_specs=[
            pl.BlockSpec((blk, H), lambda i: (i, 0)),
            pl.BlockSpec((blk, 16), lambda i: (i, 0)),
        ],
        out_shape=[
            jax.ShapeDtypeStruct((N, H), jnp.float32),
            jax.ShapeDtypeStruct((N, 16), jnp.float32),
        ],
    )(x, accx, pos_pad, accp)


def _tc2b_body(ep_ref, s2_ref):
    ep = ep_ref[...]
    ones = jnp.ones((16, 16), jnp.float32)
    inv = (ep * ep) @ ones
    s2_ref[...] = jax.nn.sigmoid(inv)


def _tc2b(ep):
    return pl.pallas_call(
        _tc2b_body,
        grid=(_TC1_GRID,),
        in_specs=[pl.BlockSpec((_TC1_BLK, 16), lambda i: (i, 0))],
        out_specs=pl.BlockSpec((_TC1_BLK, 16), lambda i: (i, 0)),
        out_shape=jax.ShapeDtypeStruct((E, 16), jnp.float32),
    )(ep)


def _sc2_body(x1_hbm, src_hbm, dst_hbm, w2_hbm, s2_hbm,
              accm_hbm,
              src_v, dst_v, rows_v, w_v, s2_v, accm_sh, sem0, sem1):
    cid = lax.axis_index("c")
    sid = lax.axis_index("s")
    wid = sid * NC + cid
    sems = (sem0, sem1)

    zero16 = jnp.zeros((16,), jnp.float32)

    def _zr(i, _):
        r = i // 8
        k = i - r * 8
        rows_v[0, r, pl.ds(k * 16, 16)] = zero16
        return 0
    lax.fori_loop(0, CHUNK * 8, _zr, 0)

    for j in range(NRCH):
        r0 = sid * RPT + j * RCH
        pltpu.sync_copy(rows_v.at[0], accm_sh.at[pl.ds(r0, RCH)])
    plsc.subcore_barrier()

    def _load(c, b):
        base = wid * EPT + c * CHUNK
        pltpu.sync_copy(src_hbm.at[pl.ds(base, CHUNK)], src_v.at[b])
        pltpu.sync_copy(dst_hbm.at[pl.ds(base, CHUNK)], dst_v.at[b])
        pltpu.async_copy(x1_hbm.at[src_v.at[b]], rows_v.at[b], sems[b])
        pltpu.async_copy(w2_hbm.at[pl.ds(base, CHUNK)], w_v.at[b], sems[b])
        pltpu.async_copy(s2_hbm.at[pl.ds(base, CHUNK)], s2_v.at[b], sems[b])

    def _drain(b):
        pltpu.make_async_copy(
            x1_hbm.at[pl.ds(0, CHUNK)], rows_v.at[b], sems[b]).wait()
        pltpu.make_async_copy(
            w2_hbm.at[pl.ds(0, CHUNK)], w_v.at[b], sems[b]).wait()
        pltpu.make_async_copy(
            s2_hbm.at[pl.ds(0, CHUNK)], s2_v.at[b], sems[b]).wait()

    _load(0, 0)

    def _outer(io, _):
        c0 = io * 2
        for b in range(2):
            c = c0 + b
            nb = 1 - b
            cn = jnp.where(c + 1 < NCHUNK, c + 1, 0)
            _load(cn, nb)
            _drain(b)

            def _edge(e, _):
                sg = s2_v[b, e, :]
                for k in range(H // 16):
                    rows_v[b, e, pl.ds(k * 16, 16)] = (
                        rows_v[b, e, pl.ds(k * 16, 16)]
                        * w_v[b, e, pl.ds(k * 16, 16)] * sg)
                return 0
            lax.fori_loop(0, CHUNK, _edge, 0)

            pltpu.sync_copy(rows_v.at[b], accm_sh.at[dst_v.at[b]], add=True)
        return 0
    lax.fori_loop(0, NCHUNK // 2, _outer, 0)
    _drain(0)

    plsc.subcore_barrier()
    for j in range(NRCH):
        r0 = sid * RPT + j * RCH
        pltpu.sync_copy(accm_sh.at[pl.ds(r0, RCH)], rows_v.at[0])
        pltpu.sync_copy(rows_v.at[0], accm_hbm.at[cid, pl.ds(r0, RCH)])


def _sc2(x1, src, dst, w2, s2):
    mesh = plsc.VectorSubcoreMesh(core_axis_name="c", subcore_axis_name="s")
    f = pl.kernel(
        _sc2_body,
        out_type=jax.ShapeDtypeStruct((NC, NP, H), jnp.float32),
        mesh=mesh,
        scratch_types=[
            pltpu.VMEM((2, CHUNK), jnp.int32),
            pltpu.VMEM((2, CHUNK), jnp.int32),
            pltpu.VMEM((2, CHUNK, H), jnp.float32),
            pltpu.VMEM((2, CHUNK, H), jnp.float32),
            pltpu.VMEM((2, CHUNK, 16), jnp.float32),
            pltpu.VMEM_SHARED((NP, H), jnp.float32),
            pltpu.SemaphoreType.DMA,
            pltpu.SemaphoreType.DMA,
        ],
    )
    return f(x1, src, dst, w2, s2)


def _tc3_body(x1_ref, accm_ref, pos1_ref, batch_ref, Wo_ref, bo_ref,
              alpha_ref, gamma_ref, beta_ref, bnw_ref, bnb_ref,
              xout_ref, posout_ref):
    x1 = x1_ref[...]
    acc = accm_ref[0, :N, :] + accm_ref[1, :N, :]
    m = acc @ Wo_ref[...] + bo_ref[...][None, :]
    x2 = x1 + m

    b = batch_ref[...]
    iot = lax.broadcasted_iota(jnp.int32, (1, B), 1)
    oh = (b == iot).astype(jnp.float32)
    ones = jnp.ones((N, 1), jnp.float32)
    cnt = jnp.maximum(
        lax.dot_general(oh, ones, (((0,), (0,)), ((), ()))), 1.0)
    mean = lax.dot_general(oh, x2, (((0,), (0,)), ((), ()))) / cnt
    xc = x2 - alpha_ref[...][None, :] * (oh @ mean)
    var = lax.dot_general(oh, xc * xc, (((0,), (0,)), ((), ()))) / cnt
    xout_ref[...] = (gamma_ref[...][None, :] * xc *
                     lax.rsqrt(oh @ var + EPS) + beta_ref[...][None, :])

    p = pos1_ref[...]
    scol = p[:, 0:1]
    smean = jnp.sum(scol) * (1.0 / N)
    svar = jnp.sum((scol - smean) ** 2) * (1.0 / N)
    v1 = p[:, 1:4]
    n1 = jnp.sum(v1 * v1) * (1.0 / (3 * N))
    v2 = p[:, 4:9]
    n2 = jnp.sum(v2 * v2) * (1.0 / (5 * N))
    sc0 = bnw_ref[0, 0] * lax.rsqrt(svar + EPS)
    off0 = bnb_ref[0, 0] - smean * sc0
    sc1 = bnw_ref[0, 1] * lax.rsqrt(n1 + EPS)
    sc2 = bnw_ref[0, 2] * lax.rsqrt(n2 + EPS)
    lane = lax.broadcasted_iota(jnp.int32, (1, 16), 1)
    scale = jnp.where(lane == 0, sc0,
                      jnp.where(lane < 4, sc1,
                                jnp.where(lane < 9, sc2, 0.0)))
    off = jnp.where(lane == 0, off0, 0.0)
    posout_ref[...] = p * scale + off


def _tc3(x1, accm, pos1, batch2, Wo, bo, alpha, gamma, beta, bnw, bnb):
    full = lambda shape: pl.BlockSpec(shape, lambda: tuple(0 for _ in shape))
    return pl.pallas_call(
        _tc3_body,
        in_specs=[
            full((N, H)), full((NC, NP, H)), full((N, 16)), full((N, 1)),
            full((H, H)), full((H,)), full((H,)), full((H,)), full((H,)),
            full((1, 3)), full((1, 1)),
        ],
        out_specs=[full((N, H)), full((N, 16))],
        out_shape=[
            jax.ShapeDtypeStruct((N, H), jnp.float32),
            jax.ShapeDtypeStruct((N, 16), jnp.float32),
        ],
    )(x1, accm, pos1, batch2, Wo, bo, alpha, gamma, beta, bnw, bnb)


def kernel(x, pos, edge_index, rbf, edge_sh, batch, W1, b1, W2, b2, Wg, bg,
           V1, c1, V2, c2, Wo, bo, alpha, gamma, beta, bn_w, bn_b):
    src = edge_index[0]
    dst = edge_index[1]
    batch2 = batch[:, None]
    pos_pad128 = _pospad(pos)

    w, ep0 = _tc1a(rbf, edge_sh, W1, b1, W2, b2, Wg, bg)
    w2 = _tc1b(rbf, V1, c1, V2, c2)
    accx, ep = _sc1(x, pos_pad128, src, dst, w, ep0)
    accp = _sc1p(dst, ep)
    s2 = _tc2b(ep)
    x1, pos1 = _tc2(x, accx, pos, accp)
    accm = _sc2(x1, src, dst, w2, s2)
    x_out, pos_out_pad = _tc3(x1, accm, pos1, batch2, Wo, bo, alpha, gamma,
                              beta, bn_w[None, :], bn_b[None, :])
    return x_out, pos_out_pad[:, :SH]
